# baseline (speedup 1.0000x reference)
import functools

import jax
import jax.numpy as jnp
from jax import lax
from jax.experimental import pallas as pl
from jax.experimental.pallas import tpu as pltpu
from jax.experimental.pallas import tpu_sc as plsc

N = 10000
E = 320000
F = 128
G = 64
N_PAD = 10240
E_PAD = 327680
NB = N_PAD // 512
HALF = F // 2
FPT = 4
EPT = E_PAD // 32
CH = 4096
CHD = 1024

_mesh = plsc.VectorSubcoreMesh(core_axis_name="c", subcore_axis_name="s")
_sc_params = pltpu.CompilerParams(needs_layout_passes=False)


@functools.partial(
    pl.kernel,
    out_type=jax.ShapeDtypeStruct((2, 16, N_PAD), jnp.float32),
    mesh=_mesh,
    scratch_types=[
        pltpu.VMEM((N_PAD,), jnp.float32),
        pltpu.VMEM((CHD,), jnp.int32),
    ],
    compiler_params=_sc_params,
)
def _sc_degree(dst_hbm, deg_out, deg_l, dbuf):
    c = lax.axis_index("c")
    s = lax.axis_index("s")
    w = c * 16 + s

    def zero(i, carry):
        deg_l[pl.ds(i * 16, 16)] = jnp.zeros((16,), jnp.float32)
        return carry
    lax.fori_loop(0, N_PAD // 16, zero, 0)

    ones16 = jnp.ones((16,), jnp.float32)

    def chunk(cb, carry):
        pltpu.sync_copy(dst_hbm.at[pl.ds(w * EPT + cb * CHD, CHD)], dbuf)

        def step(g2, c2):
            idx = dbuf[pl.ds(g2 * 16, 16)]
            plsc.addupdate_scatter(deg_l, [idx], ones16)
            return c2
        return lax.fori_loop(0, CHD // 16, step, carry)
    lax.fori_loop(0, EPT // CHD, chunk, 0)
    pltpu.sync_copy(deg_l, deg_out.at[c, s])


@functools.partial(
    pl.kernel,
    out_type=jax.ShapeDtypeStruct((2, 16, FPT, N_PAD), jnp.float32),
    mesh=_mesh,
    scratch_types=[
        pltpu.VMEM((FPT, N_PAD), jnp.float32),
        pltpu.VMEM((FPT, N_PAD), jnp.float32),
        pltpu.VMEM((CH,), jnp.int32),
        pltpu.VMEM((CH,), jnp.int32),
    ],
    compiler_params=_sc_params,
)
def _sc_spmm(u_hbm, init_hbm, src_hbm, dst_hbm, agg_out,
             u_loc, acc_loc, sbuf, dbuf):
    c = lax.axis_index("c")
    s = lax.axis_index("s")
    pltpu.sync_copy(u_hbm.at[c, s], u_loc)
    pltpu.sync_copy(init_hbm.at[c, s], acc_loc)

    ffs = [jnp.full((16,), f, jnp.int32) for f in range(FPT)]

    def chunk(cb, carry):
        base = cb * CH
        pltpu.sync_copy(src_hbm.at[pl.ds(base, CH)], sbuf)
        pltpu.sync_copy(dst_hbm.at[pl.ds(base, CH)], dbuf)

        def step(g2, c2):
            s16 = sbuf[pl.ds(g2 * 16, 16)]
            d16 = dbuf[pl.ds(g2 * 16, 16)]
            for f in range(FPT):
                vals = plsc.load_gather(u_loc, [ffs[f], s16])
                plsc.addupdate_scatter(acc_loc, [ffs[f], d16], vals)
            return c2
        return lax.fori_loop(0, CH // 16, step, carry)
    lax.fori_loop(0, E_PAD // CH, chunk, 0)
    pltpu.sync_copy(acc_loc, agg_out.at[c, s])


def _tc_pre_body(parts_ref, x_ref, w1_ref, u_ref, init_ref, dinv_ref):
    t = pl.program_id(2)
    deg_row = jnp.sum(parts_ref[...], axis=(0, 1))[None, :] + 1.0
    dinv_row = 1.0 / jnp.sqrt(deg_row)
    dinv_col = dinv_row.T
    h4 = jnp.dot(x_ref[...], w1_ref[0, 0],
                 preferred_element_type=jnp.float32)
    u4 = h4 * dinv_col
    u_ref[...] = u4.T[None, None]
    init_ref[...] = (u4 * (dinv_col * dinv_col)).T[None, None]

    @pl.when(t == 0)
    def _():
        dinv_ref[...] = dinv_col


def _tc_pre(parts, x_pad, W1):
    return pl.pallas_call(
        _tc_pre_body,
        grid=(NB, 2, 16),
        in_specs=[
            pl.BlockSpec((2, 16, 512), lambda g, c, t: (0, 0, g)),
            pl.BlockSpec((512, F), lambda g, c, t: (g, 0)),
            pl.BlockSpec((1, 1, F, FPT), lambda g, c, t: (c, t, 0, 0)),
        ],
        out_specs=[
            pl.BlockSpec((1, 1, FPT, 512), lambda g, c, t: (c, t, 0, g)),
            pl.BlockSpec((1, 1, FPT, 512), lambda g, c, t: (c, t, 0, g)),
            pl.BlockSpec((512, 1), lambda g, c, t: (g, 0)),
        ],
        out_shape=[
            jax.ShapeDtypeStruct((2, 16, FPT, N_PAD), jnp.float32),
            jax.ShapeDtypeStruct((2, 16, FPT, N_PAD), jnp.float32),
            jax.ShapeDtypeStruct((N_PAD, 1), jnp.float32),
        ],
    )(parts, x_pad, W1)


def _assemble_nm(agg_ref):
    pieces = []
    for ci in range(2):
        for ti in range(16):
            pieces.append(agg_ref[ci, ti].T)
    return jnp.concatenate(pieces, axis=1)


def _tc_mid_body(agg_ref, dinv_ref, b1_ref, w2_ref, u2_ref, init2_ref,
                 out1_s):
    t = pl.program_id(2)
    c = pl.program_id(1)
    dinv_col = dinv_ref[...]

    @pl.when((t == 0) & (c == 0))
    def _():
        agg_nm = _assemble_nm(agg_ref)
        out1_s[...] = jnp.maximum(
            agg_nm * dinv_col + b1_ref[...][None, :], 0.0)

    h4 = jnp.dot(out1_s[...], w2_ref[0, 0],
                 preferred_element_type=jnp.float32)
    u4 = h4 * dinv_col
    u2_ref[...] = u4.T[None, None]
    init2_ref[...] = (u4 * (dinv_col * dinv_col)).T[None, None]


def _tc_mid(agg1, dinv, b1, W2):
    return pl.pallas_call(
        _tc_mid_body,
        grid=(NB, 2, 16),
        in_specs=[
            pl.BlockSpec((2, 16, FPT, 512), lambda g, c, t: (0, 0, 0, g)),
            pl.BlockSpec((512, 1), lambda g, c, t: (g, 0)),
            pl.BlockSpec((F,), lambda g, c, t: (0,)),
            pl.BlockSpec((1, 1, F, FPT), lambda g, c, t: (c, t, 0, 0)),
        ],
        out_specs=[
            pl.BlockSpec((1, 1, FPT, 512), lambda g, c, t: (c, t, 0, g)),
            pl.BlockSpec((1, 1, FPT, 512), lambda g, c, t: (c, t, 0, g)),
        ],
        out_shape=[
            jax.ShapeDtypeStruct((2, 16, FPT, N_PAD), jnp.float32),
            jax.ShapeDtypeStruct((2, 16, FPT, N_PAD), jnp.float32),
        ],
        scratch_shapes=[
            pltpu.VMEM((512, F), jnp.float32),
        ],
    )(agg1, dinv, b1, W2)


def _tc_post_body(agg_ref, dinv_ref, b2_ref, batch_r_ref, batch_c_ref,
                  fc1w_ref, fc1b_ref, fc2w_ref, fc2b_ref, out_ref,
                  sum_acc, cnt_acc, max_acc):
    g = pl.program_id(0)

    @pl.when(g == 0)
    def _init():
        sum_acc[...] = jnp.zeros_like(sum_acc)
        cnt_acc[...] = jnp.zeros_like(cnt_acc)
        max_acc[...] = jnp.full_like(max_acc, -jnp.inf)

    dinv = dinv_ref[...]
    h = _assemble_nm(agg_ref)
    h = jnp.maximum(h * dinv + b2_ref[...][None, :], 0.0)
    b_row = batch_r_ref[0]
    b_col = batch_c_ref[...]
    onehot = (b_row == lax.broadcasted_iota(jnp.int32, (G, 512), 0))
    onehot = onehot.astype(jnp.float32)
    sum_acc[...] += jnp.dot(onehot, h, preferred_element_type=jnp.float32)
    cnt_acc[...] += jnp.broadcast_to(
        jnp.sum(onehot, axis=1, keepdims=True), (G, F))

    lo = jnp.min(b_col)
    hi = jnp.minimum(jnp.max(b_col), G - 1)

    def mbody(gg, carry):
        mask = (b_col == gg)
        col = jnp.max(jnp.where(mask, h, -jnp.inf), axis=0, keepdims=True)
        max_acc[pl.ds(gg, 1), :] = jnp.maximum(max_acc[pl.ds(gg, 1), :], col)
        return carry
    lax.fori_loop(lo, hi + 1, mbody, 0)

    @pl.when(g == NB - 1)
    def _fin():
        mean = sum_acc[...] / jnp.maximum(cnt_acc[...], 1.0)
        mx = max_acc[...]
        mx = jnp.where(mx == -jnp.inf, 0.0, mx)
        z = (jnp.dot(mean, fc1w_ref[0:F, :], preferred_element_type=jnp.float32)
             + jnp.dot(mx, fc1w_ref[F:2 * F, :],
                       preferred_element_type=jnp.float32)
             + fc1b_ref[...][None, :])
        z = jnp.maximum(z, 0.0)
        logits = jnp.dot(z, fc2w_ref[...],
                         preferred_element_type=jnp.float32)
        logits = logits + fc2b_ref[...][None, :]
        m = jnp.max(logits, axis=1, keepdims=True)
        lse = jnp.log(jnp.sum(jnp.exp(logits - m), axis=1, keepdims=True)) + m
        out_ref[...] = logits - lse


def _tc_post(agg2, dinv, b2, batch_r, batch_c, fc1_W, fc1_b, fc2_W, fc2_b):
    return pl.pallas_call(
        _tc_post_body,
        grid=(NB,),
        in_specs=[
            pl.BlockSpec((2, 16, FPT, 512), lambda g: (0, 0, 0, g)),
            pl.BlockSpec((512, 1), lambda g: (g, 0)),
            pl.BlockSpec((F,), lambda g: (0,)),
            pl.BlockSpec((1, 1, 512), lambda g: (g, 0, 0)),
            pl.BlockSpec((512, 1), lambda g: (g, 0)),
            pl.BlockSpec((2 * F, F), lambda g: (0, 0)),
            pl.BlockSpec((F,), lambda g: (0,)),
            pl.BlockSpec((F, 10), lambda g: (0, 0)),
            pl.BlockSpec((10,), lambda g: (0,)),
        ],
        out_specs=pl.BlockSpec((G, 10), lambda g: (0, 0)),
        out_shape=jax.ShapeDtypeStruct((G, 10), jnp.float32),
        scratch_shapes=[
            pltpu.VMEM((G, F), jnp.float32),
            pltpu.VMEM((G, F), jnp.float32),
            pltpu.VMEM((G, F), jnp.float32),
        ],
    )(agg2, dinv, b2, batch_r, batch_c, fc1_W, fc1_b, fc2_W, fc2_b)


def kernel(x, edge_index, batch, W1, b1, W2, b2, fc1_W, fc1_b, fc2_W, fc2_b):
    ei = edge_index.astype(jnp.int32)
    pad_e = E_PAD - E
    src = jnp.concatenate([ei[0], jnp.zeros((pad_e,), jnp.int32)])
    dst = jnp.concatenate([ei[1], jnp.full((pad_e,), N, jnp.int32)])
    x_pad = jnp.pad(x, ((0, N_PAD - N), (0, 0)))
    batch_pad = jnp.concatenate(
        [batch.astype(jnp.int32), jnp.full((N_PAD - N,), G, jnp.int32)])
    batch_r = batch_pad.reshape(NB, 1, 512)
    batch_c = batch_pad.reshape(N_PAD, 1)
    W1r = W1.reshape(F, 2, 16, FPT).transpose(1, 2, 0, 3)
    W2r = W2.reshape(F, 2, 16, FPT).transpose(1, 2, 0, 3)

    parts = _sc_degree(dst)
    uT, initT, dinv = _tc_pre(parts, x_pad, W1r)
    aggT1 = _sc_spmm(uT, initT, src, dst)
    u2T, init2T = _tc_mid(aggT1, dinv, b1, W2r)
    aggT2 = _sc_spmm(u2T, init2T, src, dst)
    return _tc_post(aggT2, dinv, b2, batch_r, batch_c,
                    fc1_W, fc1_b, fc2_W, fc2_b)


# speedup vs baseline: 1.1133x; 1.1133x over previous
import functools

import jax
import jax.numpy as jnp
from jax import lax
from jax.experimental import pallas as pl
from jax.experimental.pallas import tpu as pltpu
from jax.experimental.pallas import tpu_sc as plsc

N = 10000
E = 320000
F = 128
G = 64
N_PAD = 10240
E_PAD = 327680
NB = N_PAD // 512
HALF = F // 2
FPT = 4
EPT = E_PAD // 32
CH = 4096
CHD = 1024

_mesh = plsc.VectorSubcoreMesh(core_axis_name="c", subcore_axis_name="s")
_sc_params = pltpu.CompilerParams(needs_layout_passes=False)


@functools.partial(
    pl.kernel,
    out_type=jax.ShapeDtypeStruct((2, 16, N_PAD), jnp.float32),
    mesh=_mesh,
    scratch_types=[
        pltpu.VMEM((N_PAD,), jnp.float32),
        pltpu.VMEM((CHD,), jnp.int32),
    ],
    compiler_params=_sc_params,
)
def _sc_degree(dst_hbm, deg_out, deg_l, dbuf):
    c = lax.axis_index("c")
    s = lax.axis_index("s")
    w = c * 16 + s

    def zero(i, carry):
        deg_l[pl.ds(i * 16, 16)] = jnp.zeros((16,), jnp.float32)
        return carry
    lax.fori_loop(0, N_PAD // 16, zero, 0)

    ones16 = jnp.ones((16,), jnp.float32)

    def chunk(cb, carry):
        pltpu.sync_copy(dst_hbm.at[pl.ds(w * EPT + cb * CHD, CHD)], dbuf)

        def step(g2, c2):
            idx = dbuf[pl.ds(g2 * 16, 16)]
            plsc.addupdate_scatter(deg_l, [idx], ones16)
            return c2
        return lax.fori_loop(0, CHD // 16, step, carry)
    lax.fori_loop(0, EPT // CHD, chunk, 0)
    pltpu.sync_copy(deg_l, deg_out.at[c, s])


@functools.partial(
    pl.kernel,
    out_type=jax.ShapeDtypeStruct((2, 16, FPT, N_PAD), jnp.float32),
    mesh=_mesh,
    scratch_types=[
        pltpu.VMEM((FPT, N_PAD), jnp.float32),
        pltpu.VMEM((FPT, N_PAD), jnp.float32),
        pltpu.VMEM((CH,), jnp.int32),
        pltpu.VMEM((CH,), jnp.int32),
        pltpu.VMEM((CH,), jnp.int32),
        pltpu.VMEM((CH,), jnp.int32),
        pltpu.SemaphoreType.DMA,
        pltpu.SemaphoreType.DMA,
    ],
    compiler_params=_sc_params,
)
def _sc_spmm(u_hbm, init_hbm, src_hbm, dst_hbm, agg_out,
             u_loc, acc_loc, sbuf0, dbuf0, sbuf1, dbuf1, sem0, sem1):
    c = lax.axis_index("c")
    s = lax.axis_index("s")
    pltpu.sync_copy(u_hbm.at[c, s], u_loc)
    pltpu.sync_copy(init_hbm.at[c, s], acc_loc)

    ffs = [jnp.full((16,), f, jnp.int32) for f in range(FPT)]
    NCH = E_PAD // CH

    def stage(cb, sb, db, sem):
        pltpu.make_async_copy(src_hbm.at[pl.ds(cb * CH, CH)], sb, sem).start()
        pltpu.make_async_copy(dst_hbm.at[pl.ds(cb * CH, CH)], db, sem).start()

    def wait(cb, sb, db, sem):
        pltpu.make_async_copy(src_hbm.at[pl.ds(cb * CH, CH)], sb, sem).wait()
        pltpu.make_async_copy(dst_hbm.at[pl.ds(cb * CH, CH)], db, sem).wait()

    def compute(sb, db):
        def step(g2, c2):
            for k in range(4):
                off = g2 * 64 + k * 16
                s16 = sb[pl.ds(off, 16)]
                d16 = db[pl.ds(off, 16)]
                for f in range(FPT):
                    vals = plsc.load_gather(u_loc, [ffs[f], s16])
                    plsc.addupdate_scatter(acc_loc, [ffs[f], d16], vals)
            return c2
        lax.fori_loop(0, CH // 64, step, 0)

    stage(0, sbuf0, dbuf0, sem0)

    def outer(ob, carry):
        cb0 = 2 * ob
        wait(cb0, sbuf0, dbuf0, sem0)
        stage(cb0 + 1, sbuf1, dbuf1, sem1)
        compute(sbuf0, dbuf0)
        wait(cb0 + 1, sbuf1, dbuf1, sem1)

        @pl.when(ob < NCH // 2 - 1)
        def _():
            stage(cb0 + 2, sbuf0, dbuf0, sem0)
        compute(sbuf1, dbuf1)
        return carry
    lax.fori_loop(0, NCH // 2, outer, 0)
    pltpu.sync_copy(acc_loc, agg_out.at[c, s])


def _tc_pre_body(parts_ref, x_ref, w1_ref, u_ref, init_ref, dinv_ref):
    t = pl.program_id(2)
    deg_row = jnp.sum(parts_ref[...], axis=(0, 1))[None, :] + 1.0
    dinv_row = 1.0 / jnp.sqrt(deg_row)
    dinv_col = dinv_row.T
    h4 = jnp.dot(x_ref[...], w1_ref[0, 0],
                 preferred_element_type=jnp.float32)
    u4 = h4 * dinv_col
    u_ref[...] = u4.T[None, None]
    init_ref[...] = (u4 * (dinv_col * dinv_col)).T[None, None]

    @pl.when(t == 0)
    def _():
        dinv_ref[...] = dinv_col


def _tc_pre(parts, x_pad, W1):
    return pl.pallas_call(
        _tc_pre_body,
        grid=(NB, 2, 16),
        in_specs=[
            pl.BlockSpec((2, 16, 512), lambda g, c, t: (0, 0, g)),
            pl.BlockSpec((512, F), lambda g, c, t: (g, 0)),
            pl.BlockSpec((1, 1, F, FPT), lambda g, c, t: (c, t, 0, 0)),
        ],
        out_specs=[
            pl.BlockSpec((1, 1, FPT, 512), lambda g, c, t: (c, t, 0, g)),
            pl.BlockSpec((1, 1, FPT, 512), lambda g, c, t: (c, t, 0, g)),
            pl.BlockSpec((512, 1), lambda g, c, t: (g, 0)),
        ],
        out_shape=[
            jax.ShapeDtypeStruct((2, 16, FPT, N_PAD), jnp.float32),
            jax.ShapeDtypeStruct((2, 16, FPT, N_PAD), jnp.float32),
            jax.ShapeDtypeStruct((N_PAD, 1), jnp.float32),
        ],
    )(parts, x_pad, W1)


def _assemble_nm(agg_ref):
    pieces = []
    for ci in range(2):
        for ti in range(16):
            pieces.append(agg_ref[ci, ti].T)
    return jnp.concatenate(pieces, axis=1)


def _tc_mid_body(agg_ref, dinv_ref, b1_ref, w2_ref, u2_ref, init2_ref,
                 out1_s):
    t = pl.program_id(2)
    c = pl.program_id(1)
    dinv_col = dinv_ref[...]

    @pl.when((t == 0) & (c == 0))
    def _():
        agg_nm = _assemble_nm(agg_ref)
        out1_s[...] = jnp.maximum(
            agg_nm * dinv_col + b1_ref[...][None, :], 0.0)

    h4 = jnp.dot(out1_s[...], w2_ref[0, 0],
                 preferred_element_type=jnp.float32)
    u4 = h4 * dinv_col
    u2_ref[...] = u4.T[None, None]
    init2_ref[...] = (u4 * (dinv_col * dinv_col)).T[None, None]


def _tc_mid(agg1, dinv, b1, W2):
    return pl.pallas_call(
        _tc_mid_body,
        grid=(NB, 2, 16),
        in_specs=[
            pl.BlockSpec((2, 16, FPT, 512), lambda g, c, t: (0, 0, 0, g)),
            pl.BlockSpec((512, 1), lambda g, c, t: (g, 0)),
            pl.BlockSpec((F,), lambda g, c, t: (0,)),
            pl.BlockSpec((1, 1, F, FPT), lambda g, c, t: (c, t, 0, 0)),
        ],
        out_specs=[
            pl.BlockSpec((1, 1, FPT, 512), lambda g, c, t: (c, t, 0, g)),
            pl.BlockSpec((1, 1, FPT, 512), lambda g, c, t: (c, t, 0, g)),
        ],
        out_shape=[
            jax.ShapeDtypeStruct((2, 16, FPT, N_PAD), jnp.float32),
            jax.ShapeDtypeStruct((2, 16, FPT, N_PAD), jnp.float32),
        ],
        scratch_shapes=[
            pltpu.VMEM((512, F), jnp.float32),
        ],
    )(agg1, dinv, b1, W2)


def _tc_post_body(agg_ref, dinv_ref, b2_ref, batch_r_ref, batch_c_ref,
                  fc1w_ref, fc1b_ref, fc2w_ref, fc2b_ref, out_ref,
                  sum_acc, cnt_acc, max_acc):
    g = pl.program_id(0)

    @pl.when(g == 0)
    def _init():
        sum_acc[...] = jnp.zeros_like(sum_acc)
        cnt_acc[...] = jnp.zeros_like(cnt_acc)
        max_acc[...] = jnp.full_like(max_acc, -jnp.inf)

    dinv = dinv_ref[...]
    h = _assemble_nm(agg_ref)
    h = jnp.maximum(h * dinv + b2_ref[...][None, :], 0.0)
    b_row = batch_r_ref[0]
    b_col = batch_c_ref[...]
    onehot = (b_row == lax.broadcasted_iota(jnp.int32, (G, 512), 0))
    onehot = onehot.astype(jnp.float32)
    sum_acc[...] += jnp.dot(onehot, h, preferred_element_type=jnp.float32)
    cnt_acc[...] += jnp.broadcast_to(
        jnp.sum(onehot, axis=1, keepdims=True), (G, F))

    lo = jnp.min(b_col)
    hi = jnp.minimum(jnp.max(b_col), G - 1)

    def mbody(gg, carry):
        mask = (b_col == gg)
        col = jnp.max(jnp.where(mask, h, -jnp.inf), axis=0, keepdims=True)
        max_acc[pl.ds(gg, 1), :] = jnp.maximum(max_acc[pl.ds(gg, 1), :], col)
        return carry
    lax.fori_loop(lo, hi + 1, mbody, 0)

    @pl.when(g == NB - 1)
    def _fin():
        mean = sum_acc[...] / jnp.maximum(cnt_acc[...], 1.0)
        mx = max_acc[...]
        mx = jnp.where(mx == -jnp.inf, 0.0, mx)
        z = (jnp.dot(mean, fc1w_ref[0:F, :], preferred_element_type=jnp.float32)
             + jnp.dot(mx, fc1w_ref[F:2 * F, :],
                       preferred_element_type=jnp.float32)
             + fc1b_ref[...][None, :])
        z = jnp.maximum(z, 0.0)
        logits = jnp.dot(z, fc2w_ref[...],
                         preferred_element_type=jnp.float32)
        logits = logits + fc2b_ref[...][None, :]
        m = jnp.max(logits, axis=1, keepdims=True)
        lse = jnp.log(jnp.sum(jnp.exp(logits - m), axis=1, keepdims=True)) + m
        out_ref[...] = logits - lse


def _tc_post(agg2, dinv, b2, batch_r, batch_c, fc1_W, fc1_b, fc2_W, fc2_b):
    return pl.pallas_call(
        _tc_post_body,
        grid=(NB,),
        in_specs=[
            pl.BlockSpec((2, 16, FPT, 512), lambda g: (0, 0, 0, g)),
            pl.BlockSpec((512, 1), lambda g: (g, 0)),
            pl.BlockSpec((F,), lambda g: (0,)),
            pl.BlockSpec((1, 1, 512), lambda g: (g, 0, 0)),
            pl.BlockSpec((512, 1), lambda g: (g, 0)),
            pl.BlockSpec((2 * F, F), lambda g: (0, 0)),
            pl.BlockSpec((F,), lambda g: (0,)),
            pl.BlockSpec((F, 10), lambda g: (0, 0)),
            pl.BlockSpec((10,), lambda g: (0,)),
        ],
        out_specs=pl.BlockSpec((G, 10), lambda g: (0, 0)),
        out_shape=jax.ShapeDtypeStruct((G, 10), jnp.float32),
        scratch_shapes=[
            pltpu.VMEM((G, F), jnp.float32),
            pltpu.VMEM((G, F), jnp.float32),
            pltpu.VMEM((G, F), jnp.float32),
        ],
    )(agg2, dinv, b2, batch_r, batch_c, fc1_W, fc1_b, fc2_W, fc2_b)


def kernel(x, edge_index, batch, W1, b1, W2, b2, fc1_W, fc1_b, fc2_W, fc2_b):
    ei = edge_index.astype(jnp.int32)
    pad_e = E_PAD - E
    src = jnp.concatenate([ei[0], jnp.zeros((pad_e,), jnp.int32)])
    dst = jnp.concatenate([ei[1], jnp.full((pad_e,), N, jnp.int32)])
    x_pad = jnp.pad(x, ((0, N_PAD - N), (0, 0)))
    batch_pad = jnp.concatenate(
        [batch.astype(jnp.int32), jnp.full((N_PAD - N,), G, jnp.int32)])
    batch_r = batch_pad.reshape(NB, 1, 512)
    batch_c = batch_pad.reshape(N_PAD, 1)
    W1r = W1.reshape(F, 2, 16, FPT).transpose(1, 2, 0, 3)
    W2r = W2.reshape(F, 2, 16, FPT).transpose(1, 2, 0, 3)

    parts = _sc_degree(dst)
    uT, initT, dinv = _tc_pre(parts, x_pad, W1r)
    aggT1 = _sc_spmm(uT, initT, src, dst)
    u2T, init2T = _tc_mid(aggT1, dinv, b1, W2r)
    aggT2 = _sc_spmm(u2T, init2T, src, dst)
    return _tc_post(aggT2, dinv, b2, batch_r, batch_c,
                    fc1_W, fc1_b, fc2_W, fc2_b)


# speedup vs baseline: 1.1575x; 1.0397x over previous
import functools

import jax
import jax.numpy as jnp
from jax import lax
from jax.experimental import pallas as pl
from jax.experimental.pallas import tpu as pltpu
from jax.experimental.pallas import tpu_sc as plsc

N = 10000
E = 320000
F = 128
G = 64
N_PAD = 10240
E_PAD = 327680
NB = N_PAD // 512
HALF = F // 2
FPT = 4
EPT = E_PAD // 32
CH = 4096
CHD = 1024

_mesh = plsc.VectorSubcoreMesh(core_axis_name="c", subcore_axis_name="s")
_sc_params = pltpu.CompilerParams(needs_layout_passes=False)


@functools.partial(
    pl.kernel,
    out_type=jax.ShapeDtypeStruct((2, 16, N_PAD), jnp.float32),
    mesh=_mesh,
    scratch_types=[
        pltpu.VMEM((N_PAD,), jnp.float32),
        pltpu.VMEM((CHD,), jnp.int32),
    ],
    compiler_params=_sc_params,
)
def _sc_degree(dst_hbm, deg_out, deg_l, dbuf):
    c = lax.axis_index("c")
    s = lax.axis_index("s")
    w = c * 16 + s

    def zero(i, carry):
        deg_l[pl.ds(i * 16, 16)] = jnp.zeros((16,), jnp.float32)
        return carry
    lax.fori_loop(0, N_PAD // 16, zero, 0)

    ones16 = jnp.ones((16,), jnp.float32)

    def chunk(cb, carry):
        pltpu.sync_copy(dst_hbm.at[pl.ds(w * EPT + cb * CHD, CHD)], dbuf)

        def step(g2, c2):
            idx = dbuf[pl.ds(g2 * 16, 16)]
            plsc.addupdate_scatter(deg_l, [idx], ones16)
            return c2
        return lax.fori_loop(0, CHD // 16, step, carry)
    lax.fori_loop(0, EPT // CHD, chunk, 0)
    pltpu.sync_copy(deg_l, deg_out.at[c, s])


@functools.partial(
    pl.kernel,
    out_type=jax.ShapeDtypeStruct((2, 16, FPT, N_PAD), jnp.float32),
    mesh=_mesh,
    scratch_types=[
        pltpu.VMEM((N_PAD,), jnp.float32),
        pltpu.VMEM((N_PAD,), jnp.float32),
        pltpu.VMEM((N_PAD,), jnp.float32),
        pltpu.VMEM((N_PAD,), jnp.float32),
        pltpu.VMEM((N_PAD,), jnp.float32),
        pltpu.VMEM((N_PAD,), jnp.float32),
        pltpu.VMEM((N_PAD,), jnp.float32),
        pltpu.VMEM((N_PAD,), jnp.float32),
        pltpu.VMEM((CH,), jnp.int32),
        pltpu.VMEM((CH,), jnp.int32),
        pltpu.VMEM((CH,), jnp.int32),
        pltpu.VMEM((CH,), jnp.int32),
        pltpu.SemaphoreType.DMA,
        pltpu.SemaphoreType.DMA,
    ],
    compiler_params=_sc_params,
)
def _sc_spmm(u_hbm, init_hbm, src_hbm, dst_hbm, agg_out,
             u0, u1, u2, u3, a0, a1, a2, a3,
             sbuf0, dbuf0, sbuf1, dbuf1, sem0, sem1):
    c = lax.axis_index("c")
    s = lax.axis_index("s")
    us = [u0, u1, u2, u3]
    accs = [a0, a1, a2, a3]
    for f in range(FPT):
        pltpu.sync_copy(u_hbm.at[c, s, f], us[f])
        pltpu.sync_copy(init_hbm.at[c, s, f], accs[f])

    NCH = E_PAD // CH

    def stage(cb, sb, db, sem):
        pltpu.make_async_copy(src_hbm.at[pl.ds(cb * CH, CH)], sb, sem).start()
        pltpu.make_async_copy(dst_hbm.at[pl.ds(cb * CH, CH)], db, sem).start()

    def wait(cb, sb, db, sem):
        pltpu.make_async_copy(src_hbm.at[pl.ds(cb * CH, CH)], sb, sem).wait()
        pltpu.make_async_copy(dst_hbm.at[pl.ds(cb * CH, CH)], db, sem).wait()

    def compute(sb, db):
        def step(g2, c2):
            for k in range(4):
                off = g2 * 64 + k * 16
                s16 = sb[pl.ds(off, 16)]
                d16 = db[pl.ds(off, 16)]
                for f in range(FPT):
                    vals = plsc.load_gather(us[f], [s16])
                    plsc.addupdate_scatter(accs[f], [d16], vals)
            return c2
        lax.fori_loop(0, CH // 64, step, 0)

    stage(0, sbuf0, dbuf0, sem0)

    def outer(ob, carry):
        cb0 = 2 * ob
        wait(cb0, sbuf0, dbuf0, sem0)
        stage(cb0 + 1, sbuf1, dbuf1, sem1)
        compute(sbuf0, dbuf0)
        wait(cb0 + 1, sbuf1, dbuf1, sem1)

        @pl.when(ob < NCH // 2 - 1)
        def _():
            stage(cb0 + 2, sbuf0, dbuf0, sem0)
        compute(sbuf1, dbuf1)
        return carry
    lax.fori_loop(0, NCH // 2, outer, 0)
    for f in range(FPT):
        pltpu.sync_copy(accs[f], agg_out.at[c, s, f])


def _tc_pre_body(parts_ref, x_ref, w1_ref, u_ref, init_ref, dinv_ref):
    t = pl.program_id(2)
    deg_row = jnp.sum(parts_ref[...], axis=(0, 1))[None, :] + 1.0
    dinv_row = 1.0 / jnp.sqrt(deg_row)
    dinv_col = dinv_row.T
    h4 = jnp.dot(x_ref[...], w1_ref[0, 0],
                 preferred_element_type=jnp.float32)
    u4 = h4 * dinv_col
    u_ref[...] = u4.T[None, None]
    init_ref[...] = (u4 * (dinv_col * dinv_col)).T[None, None]

    @pl.when(t == 0)
    def _():
        dinv_ref[...] = dinv_col


def _tc_pre(parts, x_pad, W1):
    return pl.pallas_call(
        _tc_pre_body,
        grid=(NB, 2, 16),
        in_specs=[
            pl.BlockSpec((2, 16, 512), lambda g, c, t: (0, 0, g)),
            pl.BlockSpec((512, F), lambda g, c, t: (g, 0)),
            pl.BlockSpec((1, 1, F, FPT), lambda g, c, t: (c, t, 0, 0)),
        ],
        out_specs=[
            pl.BlockSpec((1, 1, FPT, 512), lambda g, c, t: (c, t, 0, g)),
            pl.BlockSpec((1, 1, FPT, 512), lambda g, c, t: (c, t, 0, g)),
            pl.BlockSpec((512, 1), lambda g, c, t: (g, 0)),
        ],
        out_shape=[
            jax.ShapeDtypeStruct((2, 16, FPT, N_PAD), jnp.float32),
            jax.ShapeDtypeStruct((2, 16, FPT, N_PAD), jnp.float32),
            jax.ShapeDtypeStruct((N_PAD, 1), jnp.float32),
        ],
    )(parts, x_pad, W1)


def _assemble_nm(agg_ref):
    pieces = []
    for ci in range(2):
        for ti in range(16):
            pieces.append(agg_ref[ci, ti].T)
    return jnp.concatenate(pieces, axis=1)


def _tc_mid_body(agg_ref, dinv_ref, b1_ref, w2_ref, u2_ref, init2_ref,
                 out1_s):
    t = pl.program_id(2)
    c = pl.program_id(1)
    dinv_col = dinv_ref[...]

    @pl.when((t == 0) & (c == 0))
    def _():
        agg_nm = _assemble_nm(agg_ref)
        out1_s[...] = jnp.maximum(
            agg_nm * dinv_col + b1_ref[...][None, :], 0.0)

    h4 = jnp.dot(out1_s[...], w2_ref[0, 0],
                 preferred_element_type=jnp.float32)
    u4 = h4 * dinv_col
    u2_ref[...] = u4.T[None, None]
    init2_ref[...] = (u4 * (dinv_col * dinv_col)).T[None, None]


def _tc_mid(agg1, dinv, b1, W2):
    return pl.pallas_call(
        _tc_mid_body,
        grid=(NB, 2, 16),
        in_specs=[
            pl.BlockSpec((2, 16, FPT, 512), lambda g, c, t: (0, 0, 0, g)),
            pl.BlockSpec((512, 1), lambda g, c, t: (g, 0)),
            pl.BlockSpec((F,), lambda g, c, t: (0,)),
            pl.BlockSpec((1, 1, F, FPT), lambda g, c, t: (c, t, 0, 0)),
        ],
        out_specs=[
            pl.BlockSpec((1, 1, FPT, 512), lambda g, c, t: (c, t, 0, g)),
            pl.BlockSpec((1, 1, FPT, 512), lambda g, c, t: (c, t, 0, g)),
        ],
        out_shape=[
            jax.ShapeDtypeStruct((2, 16, FPT, N_PAD), jnp.float32),
            jax.ShapeDtypeStruct((2, 16, FPT, N_PAD), jnp.float32),
        ],
        scratch_shapes=[
            pltpu.VMEM((512, F), jnp.float32),
        ],
    )(agg1, dinv, b1, W2)


def _tc_post_body(agg_ref, dinv_ref, b2_ref, batch_r_ref, batch_c_ref,
                  fc1w_ref, fc1b_ref, fc2w_ref, fc2b_ref, out_ref,
                  sum_acc, cnt_acc, max_acc):
    g = pl.program_id(0)

    @pl.when(g == 0)
    def _init():
        sum_acc[...] = jnp.zeros_like(sum_acc)
        cnt_acc[...] = jnp.zeros_like(cnt_acc)
        max_acc[...] = jnp.full_like(max_acc, -jnp.inf)

    dinv = dinv_ref[...]
    h = _assemble_nm(agg_ref)
    h = jnp.maximum(h * dinv + b2_ref[...][None, :], 0.0)
    b_row = batch_r_ref[0]
    b_col = batch_c_ref[...]
    onehot = (b_row == lax.broadcasted_iota(jnp.int32, (G, 512), 0))
    onehot = onehot.astype(jnp.float32)
    sum_acc[...] += jnp.dot(onehot, h, preferred_element_type=jnp.float32)
    cnt_acc[...] += jnp.broadcast_to(
        jnp.sum(onehot, axis=1, keepdims=True), (G, F))

    lo = jnp.min(b_col)
    hi = jnp.minimum(jnp.max(b_col), G - 1)

    def mbody(gg, carry):
        mask = (b_col == gg)
        col = jnp.max(jnp.where(mask, h, -jnp.inf), axis=0, keepdims=True)
        max_acc[pl.ds(gg, 1), :] = jnp.maximum(max_acc[pl.ds(gg, 1), :], col)
        return carry
    lax.fori_loop(lo, hi + 1, mbody, 0)

    @pl.when(g == NB - 1)
    def _fin():
        mean = sum_acc[...] / jnp.maximum(cnt_acc[...], 1.0)
        mx = max_acc[...]
        mx = jnp.where(mx == -jnp.inf, 0.0, mx)
        z = (jnp.dot(mean, fc1w_ref[0:F, :], preferred_element_type=jnp.float32)
             + jnp.dot(mx, fc1w_ref[F:2 * F, :],
                       preferred_element_type=jnp.float32)
             + fc1b_ref[...][None, :])
        z = jnp.maximum(z, 0.0)
        logits = jnp.dot(z, fc2w_ref[...],
                         preferred_element_type=jnp.float32)
        logits = logits + fc2b_ref[...][None, :]
        m = jnp.max(logits, axis=1, keepdims=True)
        lse = jnp.log(jnp.sum(jnp.exp(logits - m), axis=1, keepdims=True)) + m
        out_ref[...] = logits - lse


def _tc_post(agg2, dinv, b2, batch_r, batch_c, fc1_W, fc1_b, fc2_W, fc2_b):
    return pl.pallas_call(
        _tc_post_body,
        grid=(NB,),
        in_specs=[
            pl.BlockSpec((2, 16, FPT, 512), lambda g: (0, 0, 0, g)),
            pl.BlockSpec((512, 1), lambda g: (g, 0)),
            pl.BlockSpec((F,), lambda g: (0,)),
            pl.BlockSpec((1, 1, 512), lambda g: (g, 0, 0)),
            pl.BlockSpec((512, 1), lambda g: (g, 0)),
            pl.BlockSpec((2 * F, F), lambda g: (0, 0)),
            pl.BlockSpec((F,), lambda g: (0,)),
            pl.BlockSpec((F, 10), lambda g: (0, 0)),
            pl.BlockSpec((10,), lambda g: (0,)),
        ],
        out_specs=pl.BlockSpec((G, 10), lambda g: (0, 0)),
        out_shape=jax.ShapeDtypeStruct((G, 10), jnp.float32),
        scratch_shapes=[
            pltpu.VMEM((G, F), jnp.float32),
            pltpu.VMEM((G, F), jnp.float32),
            pltpu.VMEM((G, F), jnp.float32),
        ],
    )(agg2, dinv, b2, batch_r, batch_c, fc1_W, fc1_b, fc2_W, fc2_b)


def kernel(x, edge_index, batch, W1, b1, W2, b2, fc1_W, fc1_b, fc2_W, fc2_b):
    ei = edge_index.astype(jnp.int32)
    pad_e = E_PAD - E
    src = jnp.concatenate([ei[0], jnp.zeros((pad_e,), jnp.int32)])
    dst = jnp.concatenate([ei[1], jnp.full((pad_e,), N, jnp.int32)])
    x_pad = jnp.pad(x, ((0, N_PAD - N), (0, 0)))
    batch_pad = jnp.concatenate(
        [batch.astype(jnp.int32), jnp.full((N_PAD - N,), G, jnp.int32)])
    batch_r = batch_pad.reshape(NB, 1, 512)
    batch_c = batch_pad.reshape(N_PAD, 1)
    W1r = W1.reshape(F, 2, 16, FPT).transpose(1, 2, 0, 3)
    W2r = W2.reshape(F, 2, 16, FPT).transpose(1, 2, 0, 3)

    parts = _sc_degree(dst)
    uT, initT, dinv = _tc_pre(parts, x_pad, W1r)
    aggT1 = _sc_spmm(uT, initT, src, dst)
    u2T, init2T = _tc_mid(aggT1, dinv, b1, W2r)
    aggT2 = _sc_spmm(u2T, init2T, src, dst)
    return _tc_post(aggT2, dinv, b2, batch_r, batch_c,
                    fc1_W, fc1_b, fc2_W, fc2_b)


# speedup vs baseline: 1.1793x; 1.0189x over previous
import functools

import jax
import jax.numpy as jnp
from jax import lax
from jax.experimental import pallas as pl
from jax.experimental.pallas import tpu as pltpu
from jax.experimental.pallas import tpu_sc as plsc

N = 10000
E = 320000
F = 128
G = 64
N_PAD = 10240
E_PAD = 327680
NB = N_PAD // 512
HALF = F // 2
FPT = 4
EPT = E_PAD // 32
CH = 4096
CHD = 1024

_mesh = plsc.VectorSubcoreMesh(core_axis_name="c", subcore_axis_name="s")
_sc_params = pltpu.CompilerParams(needs_layout_passes=False)


@functools.partial(
    pl.kernel,
    out_type=jax.ShapeDtypeStruct((2, 16, N_PAD), jnp.float32),
    mesh=_mesh,
    scratch_types=[
        pltpu.VMEM((N_PAD,), jnp.float32),
        pltpu.VMEM((CHD,), jnp.int32),
    ],
    compiler_params=_sc_params,
)
def _sc_degree(dst_hbm, deg_out, deg_l, dbuf):
    c = lax.axis_index("c")
    s = lax.axis_index("s")
    w = c * 16 + s

    def zero(i, carry):
        deg_l[pl.ds(i * 16, 16)] = jnp.zeros((16,), jnp.float32)
        return carry
    lax.fori_loop(0, N_PAD // 16, zero, 0)

    ones16 = jnp.ones((16,), jnp.float32)

    def chunk(cb, carry):
        pltpu.sync_copy(dst_hbm.at[pl.ds(w * EPT + cb * CHD, CHD)], dbuf)

        def step(g2, c2):
            idx = dbuf[pl.ds(g2 * 16, 16)]
            plsc.addupdate_scatter(deg_l, [idx], ones16)
            return c2
        return lax.fori_loop(0, CHD // 16, step, carry)
    lax.fori_loop(0, EPT // CHD, chunk, 0)
    pltpu.sync_copy(deg_l, deg_out.at[c, s])


@functools.partial(
    pl.kernel,
    out_type=jax.ShapeDtypeStruct((2, 16, FPT, N_PAD), jnp.float32),
    mesh=_mesh,
    scratch_types=[
        pltpu.VMEM((N_PAD,), jnp.float32),
        pltpu.VMEM((N_PAD,), jnp.float32),
        pltpu.VMEM((N_PAD,), jnp.float32),
        pltpu.VMEM((N_PAD,), jnp.float32),
        pltpu.VMEM((N_PAD,), jnp.float32),
        pltpu.VMEM((N_PAD,), jnp.float32),
        pltpu.VMEM((N_PAD,), jnp.float32),
        pltpu.VMEM((N_PAD,), jnp.float32),
        pltpu.VMEM((CH,), jnp.int32),
        pltpu.VMEM((CH,), jnp.int32),
        pltpu.VMEM((CH,), jnp.int32),
        pltpu.VMEM((CH,), jnp.int32),
        pltpu.SemaphoreType.DMA,
        pltpu.SemaphoreType.DMA,
    ],
    compiler_params=_sc_params,
)
def _sc_spmm(u_hbm, init_hbm, src_hbm, dst_hbm, agg_out,
             u0, u1, u2, u3, a0, a1, a2, a3,
             sbuf0, dbuf0, sbuf1, dbuf1, sem0, sem1):
    c = lax.axis_index("c")
    s = lax.axis_index("s")
    us = [u0, u1, u2, u3]
    accs = [a0, a1, a2, a3]
    for f in range(FPT):
        pltpu.sync_copy(u_hbm.at[c, s, f], us[f])
        pltpu.sync_copy(init_hbm.at[c, s, f], accs[f])

    NCH = E_PAD // CH

    def stage(cb, sb, db, sem):
        pltpu.make_async_copy(src_hbm.at[pl.ds(cb * CH, CH)], sb, sem).start()
        pltpu.make_async_copy(dst_hbm.at[pl.ds(cb * CH, CH)], db, sem).start()

    def wait(cb, sb, db, sem):
        pltpu.make_async_copy(src_hbm.at[pl.ds(cb * CH, CH)], sb, sem).wait()
        pltpu.make_async_copy(dst_hbm.at[pl.ds(cb * CH, CH)], db, sem).wait()

    def compute(sb, db):
        def step(g2, c2):
            for k in range(4):
                off = g2 * 64 + k * 16
                s16 = sb[pl.ds(off, 16)]
                d16 = db[pl.ds(off, 16)]
                for f in range(FPT):
                    vals = plsc.load_gather(us[f], [s16])
                    plsc.addupdate_scatter(accs[f], [d16], vals)
            return c2
        lax.fori_loop(0, CH // 64, step, 0)

    stage(0, sbuf0, dbuf0, sem0)

    def outer(ob, carry):
        cb0 = 2 * ob
        wait(cb0, sbuf0, dbuf0, sem0)
        stage(cb0 + 1, sbuf1, dbuf1, sem1)
        compute(sbuf0, dbuf0)
        wait(cb0 + 1, sbuf1, dbuf1, sem1)

        @pl.when(ob < NCH // 2 - 1)
        def _():
            stage(cb0 + 2, sbuf0, dbuf0, sem0)
        compute(sbuf1, dbuf1)
        return carry
    lax.fori_loop(0, NCH // 2, outer, 0)
    for f in range(FPT):
        pltpu.sync_copy(accs[f], agg_out.at[c, s, f])


def _tc_pre_body(parts_ref, x_ref, w1_ref, u_ref, init_ref, dinv_ref,
                 dinv_s):
    t = pl.program_id(2)
    c = pl.program_id(1)

    @pl.when((t == 0) & (c == 0))
    def _():
        deg_row = jnp.sum(parts_ref[...], axis=(0, 1))[None, :] + 1.0
        dinv_col = (1.0 / jnp.sqrt(deg_row)).T
        dinv_s[...] = dinv_col
        dinv_ref[...] = dinv_col

    dinv_col = dinv_s[...]
    h4 = jnp.dot(x_ref[...], w1_ref[0, 0],
                 preferred_element_type=jnp.float32)
    u4 = h4 * dinv_col
    u_ref[...] = u4.T[None, None]
    init_ref[...] = (u4 * (dinv_col * dinv_col)).T[None, None]


def _tc_pre(parts, x_pad, W1):
    return pl.pallas_call(
        _tc_pre_body,
        grid=(NB, 2, 16),
        in_specs=[
            pl.BlockSpec((2, 16, 512), lambda g, c, t: (0, 0, g)),
            pl.BlockSpec((512, F), lambda g, c, t: (g, 0)),
            pl.BlockSpec((1, 1, F, FPT), lambda g, c, t: (c, t, 0, 0)),
        ],
        out_specs=[
            pl.BlockSpec((1, 1, FPT, 512), lambda g, c, t: (c, t, 0, g)),
            pl.BlockSpec((1, 1, FPT, 512), lambda g, c, t: (c, t, 0, g)),
            pl.BlockSpec((512, 1), lambda g, c, t: (g, 0)),
        ],
        out_shape=[
            jax.ShapeDtypeStruct((2, 16, FPT, N_PAD), jnp.float32),
            jax.ShapeDtypeStruct((2, 16, FPT, N_PAD), jnp.float32),
            jax.ShapeDtypeStruct((N_PAD, 1), jnp.float32),
        ],
        scratch_shapes=[pltpu.VMEM((512, 1), jnp.float32)],
    )(parts, x_pad, W1)


def _assemble_nm(agg_ref):
    pieces = []
    for ci in range(2):
        for ti in range(16):
            pieces.append(agg_ref[ci, ti].T)
    return jnp.concatenate(pieces, axis=1)


def _tc_mid_body(agg_ref, dinv_ref, b1_ref, w2_ref, u2_ref, init2_ref,
                 out1_s):
    t = pl.program_id(2)
    c = pl.program_id(1)
    dinv_col = dinv_ref[...]

    @pl.when((t == 0) & (c == 0))
    def _():
        agg_nm = _assemble_nm(agg_ref)
        out1_s[...] = jnp.maximum(
            agg_nm * dinv_col + b1_ref[...][None, :], 0.0)

    h4 = jnp.dot(out1_s[...], w2_ref[0, 0],
                 preferred_element_type=jnp.float32)
    u4 = h4 * dinv_col
    u2_ref[...] = u4.T[None, None]
    init2_ref[...] = (u4 * (dinv_col * dinv_col)).T[None, None]


def _tc_mid(agg1, dinv, b1, W2):
    return pl.pallas_call(
        _tc_mid_body,
        grid=(NB, 2, 16),
        in_specs=[
            pl.BlockSpec((2, 16, FPT, 512), lambda g, c, t: (0, 0, 0, g)),
            pl.BlockSpec((512, 1), lambda g, c, t: (g, 0)),
            pl.BlockSpec((F,), lambda g, c, t: (0,)),
            pl.BlockSpec((1, 1, F, FPT), lambda g, c, t: (c, t, 0, 0)),
        ],
        out_specs=[
            pl.BlockSpec((1, 1, FPT, 512), lambda g, c, t: (c, t, 0, g)),
            pl.BlockSpec((1, 1, FPT, 512), lambda g, c, t: (c, t, 0, g)),
        ],
        out_shape=[
            jax.ShapeDtypeStruct((2, 16, FPT, N_PAD), jnp.float32),
            jax.ShapeDtypeStruct((2, 16, FPT, N_PAD), jnp.float32),
        ],
        scratch_shapes=[
            pltpu.VMEM((512, F), jnp.float32),
        ],
    )(agg1, dinv, b1, W2)


def _tc_post_body(agg_ref, dinv_ref, b2_ref, batch_r_ref, batch_c_ref,
                  fc1w_ref, fc1b_ref, fc2w_ref, fc2b_ref, out_ref,
                  sum_acc, cnt_acc, max_acc):
    g = pl.program_id(0)

    @pl.when(g == 0)
    def _init():
        sum_acc[...] = jnp.zeros_like(sum_acc)
        cnt_acc[...] = jnp.zeros_like(cnt_acc)
        max_acc[...] = jnp.full_like(max_acc, -jnp.inf)

    dinv = dinv_ref[...]
    h = _assemble_nm(agg_ref)
    h = jnp.maximum(h * dinv + b2_ref[...][None, :], 0.0)
    b_row = batch_r_ref[0]
    b_col = batch_c_ref[...]
    onehot = (b_row == lax.broadcasted_iota(jnp.int32, (G, 512), 0))
    onehot = onehot.astype(jnp.float32)
    sum_acc[...] += jnp.dot(onehot, h, preferred_element_type=jnp.float32)
    cnt_acc[...] += jnp.broadcast_to(
        jnp.sum(onehot, axis=1, keepdims=True), (G, F))

    lo = jnp.min(b_col)
    hi = jnp.minimum(jnp.max(b_col), G - 1)

    def mbody(gg, carry):
        mask = (b_col == gg)
        col = jnp.max(jnp.where(mask, h, -jnp.inf), axis=0, keepdims=True)
        max_acc[pl.ds(gg, 1), :] = jnp.maximum(max_acc[pl.ds(gg, 1), :], col)
        return carry
    lax.fori_loop(lo, hi + 1, mbody, 0)

    @pl.when(g == NB - 1)
    def _fin():
        mean = sum_acc[...] / jnp.maximum(cnt_acc[...], 1.0)
        mx = max_acc[...]
        mx = jnp.where(mx == -jnp.inf, 0.0, mx)
        z = (jnp.dot(mean, fc1w_ref[0:F, :], preferred_element_type=jnp.float32)
             + jnp.dot(mx, fc1w_ref[F:2 * F, :],
                       preferred_element_type=jnp.float32)
             + fc1b_ref[...][None, :])
        z = jnp.maximum(z, 0.0)
        logits = jnp.dot(z, fc2w_ref[...],
                         preferred_element_type=jnp.float32)
        logits = logits + fc2b_ref[...][None, :]
        m = jnp.max(logits, axis=1, keepdims=True)
        lse = jnp.log(jnp.sum(jnp.exp(logits - m), axis=1, keepdims=True)) + m
        out_ref[...] = logits - lse


def _tc_post(agg2, dinv, b2, batch_r, batch_c, fc1_W, fc1_b, fc2_W, fc2_b):
    return pl.pallas_call(
        _tc_post_body,
        grid=(NB,),
        in_specs=[
            pl.BlockSpec((2, 16, FPT, 512), lambda g: (0, 0, 0, g)),
            pl.BlockSpec((512, 1), lambda g: (g, 0)),
            pl.BlockSpec((F,), lambda g: (0,)),
            pl.BlockSpec((1, 1, 512), lambda g: (g, 0, 0)),
            pl.BlockSpec((512, 1), lambda g: (g, 0)),
            pl.BlockSpec((2 * F, F), lambda g: (0, 0)),
            pl.BlockSpec((F,), lambda g: (0,)),
            pl.BlockSpec((F, 10), lambda g: (0, 0)),
            pl.BlockSpec((10,), lambda g: (0,)),
        ],
        out_specs=pl.BlockSpec((G, 10), lambda g: (0, 0)),
        out_shape=jax.ShapeDtypeStruct((G, 10), jnp.float32),
        scratch_shapes=[
            pltpu.VMEM((G, F), jnp.float32),
            pltpu.VMEM((G, F), jnp.float32),
            pltpu.VMEM((G, F), jnp.float32),
        ],
    )(agg2, dinv, b2, batch_r, batch_c, fc1_W, fc1_b, fc2_W, fc2_b)


def kernel(x, edge_index, batch, W1, b1, W2, b2, fc1_W, fc1_b, fc2_W, fc2_b):
    ei = edge_index.astype(jnp.int32)
    pad_e = E_PAD - E
    src = jnp.concatenate([ei[0], jnp.zeros((pad_e,), jnp.int32)])
    dst = jnp.concatenate([ei[1], jnp.full((pad_e,), N, jnp.int32)])
    x_pad = jnp.pad(x, ((0, N_PAD - N), (0, 0)))
    batch_pad = jnp.concatenate(
        [batch.astype(jnp.int32), jnp.full((N_PAD - N,), G, jnp.int32)])
    batch_r = batch_pad.reshape(NB, 1, 512)
    batch_c = batch_pad.reshape(N_PAD, 1)
    W1r = W1.reshape(F, 2, 16, FPT).transpose(1, 2, 0, 3)
    W2r = W2.reshape(F, 2, 16, FPT).transpose(1, 2, 0, 3)

    parts = _sc_degree(dst)
    uT, initT, dinv = _tc_pre(parts, x_pad, W1r)
    aggT1 = _sc_spmm(uT, initT, src, dst)
    u2T, init2T = _tc_mid(aggT1, dinv, b1, W2r)
    aggT2 = _sc_spmm(u2T, init2T, src, dst)
    return _tc_post(aggT2, dinv, b2, batch_r, batch_c,
                    fc1_W, fc1_b, fc2_W, fc2_b)
